# baseline (speedup 1.0000x reference)
import functools

import jax
import jax.numpy as jnp
from jax import lax
from jax.experimental import pallas as pl
from jax.experimental.pallas import tpu as pltpu
from jax.experimental.pallas import tpu_sc as plsc

B = 16384
D = 64
L = 16
NC = 2
NS = 16
NW = NC * NS
BPW = B // NW
CH = 64
NSB = BPW // CH

_mesh = plsc.VectorSubcoreMesh(core_axis_name="c", subcore_axis_name="s")


@functools.partial(
    pl.kernel,
    mesh=_mesh,
    compiler_params=pltpu.CompilerParams(needs_layout_passes=False),
    out_type=jax.ShapeDtypeStruct((B,), jnp.float32),
    scratch_types=[
        pltpu.VMEM((NSB, CH), jnp.int32),
        pltpu.VMEM((NSB, CH), jnp.int32),
        pltpu.VMEM((CH // 8, 8, D), jnp.float32),
        pltpu.VMEM((CH // 8, 8, D), jnp.float32),
        pltpu.VMEM((CH // 8, 8, D), jnp.float32),
        pltpu.VMEM((CH // 8, 8, D), jnp.float32),
        pltpu.VMEM((BPW,), jnp.float32),
        pltpu.SemaphoreType.DMA,
        pltpu.SemaphoreType.DMA,
        pltpu.SemaphoreType.DMA,
        pltpu.SemaphoreType.DMA,
        pltpu.SemaphoreType.DMA,
    ],
)
def _mf_sc(user_hbm, item_hbm, utab_hbm, itab_hbm, out_hbm,
           uidx, iidx, ub0, ub1, ib0, ib1, oacc, us0, us1, is0, is1, isem):
    wid = lax.axis_index("s") * NC + lax.axis_index("c")
    base = wid * BPW
    ubufs, ibufs = (ub0, ub1), (ib0, ib1)
    usems, isems = (us0, us1), (is0, is1)

    idx_copies = []
    for j in range(NSB):
        idx_copies.append(pltpu.async_copy(
            user_hbm.at[pl.ds(base + j * CH, CH)], uidx.at[j], isem))
        idx_copies.append(pltpu.async_copy(
            item_hbm.at[pl.ds(base + j * CH, CH)], iidx.at[j], isem))
    for c in idx_copies:
        c.wait()

    def fire(s):
        p = s % 2

        def g_body(g, carry):
            uvec = uidx[s, pl.ds(g * L, L)]
            ivec = iidx[s, pl.ds(g * L, L)]
            for k in range(L):
                m = g * L + k
                ur, ir = uvec[k], ivec[k]
                pltpu.async_copy(
                    utab_hbm.at[pl.ds(ur >> 3, 1), ur & 7, :],
                    ubufs[p].at[pl.ds(m // 8, 1), m % 8, :], usems[p])
                pltpu.async_copy(
                    itab_hbm.at[pl.ds(ir >> 3, 1), ir & 7, :],
                    ibufs[p].at[pl.ds(m // 8, 1), m % 8, :], isems[p])
            return carry

        lax.fori_loop(0, CH // L, g_body, 0)

    def drain(s):
        p = s % 2
        pltpu.make_async_copy(utab_hbm.at[pl.ds(0, CH // 8), :, :], ubufs[p],
                              usems[p]).wait()
        pltpu.make_async_copy(itab_hbm.at[pl.ds(0, CH // 8), :, :], ibufs[p],
                              isems[p]).wait()

    lane = lax.broadcasted_iota(jnp.int32, (L,), 0)

    fire(0)
    for s in range(NSB):
        drain(s)
        if s + 1 < NSB:
            fire(s + 1)
        ub, ib = ubufs[s % 2], ibufs[s % 2]

        def group_body(g, carry, ub=ub, ib=ib, s=s):
            acc = jnp.zeros((L,), jnp.float32)
            for k in range(L):
                b = g * L + k
                p = jnp.zeros((L,), jnp.float32)
                for f in range(0, D, L):
                    u = ub[b // 8, b % 8, pl.ds(f, L)]
                    v = ib[b // 8, b % 8, pl.ds(f, L)]
                    p = p + u * v
                acc = jnp.where(lane == k, jnp.sum(p), acc)
            oacc[pl.ds(s * CH + g * L, L)] = acc
            return carry

        lax.fori_loop(0, CH // L, group_body, 0)

    pltpu.sync_copy(oacc, out_hbm.at[pl.ds(base, BPW)])


def kernel(user, item, user_factors, item_factors):
    utab = user_factors.reshape(12500, 8, D)
    itab = item_factors.reshape(12500, 8, D)
    return _mf_sc(user.astype(jnp.int32), item.astype(jnp.int32), utab, itab)


# speedup vs baseline: 1.0833x; 1.0833x over previous
import functools

import jax
import jax.numpy as jnp
from jax import lax
from jax.experimental import pallas as pl
from jax.experimental.pallas import tpu as pltpu
from jax.experimental.pallas import tpu_sc as plsc

B = 16384
D = 64
L = 16
NC = 2
NS = 16
NW = NC * NS
BPW = B // NW
CH = 128
NSB = BPW // CH

_mesh = plsc.VectorSubcoreMesh(core_axis_name="c", subcore_axis_name="s")


@functools.partial(
    pl.kernel,
    mesh=_mesh,
    compiler_params=pltpu.CompilerParams(needs_layout_passes=False),
    out_type=jax.ShapeDtypeStruct((B,), jnp.float32),
    scratch_types=[
        pltpu.VMEM((NSB, CH), jnp.int32),
        pltpu.VMEM((NSB, CH), jnp.int32),
        pltpu.VMEM((CH // 8, 8, D), jnp.float32),
        pltpu.VMEM((CH // 8, 8, D), jnp.float32),
        pltpu.VMEM((CH // 8, 8, D), jnp.float32),
        pltpu.VMEM((CH // 8, 8, D), jnp.float32),
        pltpu.VMEM((BPW,), jnp.float32),
        pltpu.SemaphoreType.DMA,
        pltpu.SemaphoreType.DMA,
        pltpu.SemaphoreType.DMA,
        pltpu.SemaphoreType.DMA,
        pltpu.SemaphoreType.DMA,
    ],
)
def _mf_sc(user_hbm, item_hbm, utab_hbm, itab_hbm, out_hbm,
           uidx, iidx, ub0, ub1, ib0, ib1, oacc, us0, us1, is0, is1, isem):
    wid = lax.axis_index("s") * NC + lax.axis_index("c")
    base = wid * BPW
    ubufs, ibufs = (ub0, ub1), (ib0, ib1)
    usems, isems = (us0, us1), (is0, is1)

    idx_copies = []
    for j in range(NSB):
        idx_copies.append(pltpu.async_copy(
            user_hbm.at[pl.ds(base + j * CH, CH)], uidx.at[j], isem))
        idx_copies.append(pltpu.async_copy(
            item_hbm.at[pl.ds(base + j * CH, CH)], iidx.at[j], isem))
    for c in idx_copies:
        c.wait()

    def fire(s):
        p = s % 2

        def g_body(g, carry):
            uvec = uidx[s, pl.ds(g * L, L)]
            ivec = iidx[s, pl.ds(g * L, L)]
            for k in range(L):
                m = g * L + k
                ur, ir = uvec[k], ivec[k]
                pltpu.async_copy(
                    utab_hbm.at[pl.ds(ur >> 3, 1), ur & 7, :],
                    ubufs[p].at[pl.ds(m // 8, 1), m % 8, :], usems[p])
                pltpu.async_copy(
                    itab_hbm.at[pl.ds(ir >> 3, 1), ir & 7, :],
                    ibufs[p].at[pl.ds(m // 8, 1), m % 8, :], isems[p])
            return carry

        lax.fori_loop(0, CH // L, g_body, 0)

    def drain(s):
        p = s % 2
        pltpu.make_async_copy(utab_hbm.at[pl.ds(0, CH // 8), :, :], ubufs[p],
                              usems[p]).wait()
        pltpu.make_async_copy(itab_hbm.at[pl.ds(0, CH // 8), :, :], ibufs[p],
                              isems[p]).wait()

    lane = lax.broadcasted_iota(jnp.int32, (L,), 0)

    fire(0)
    for s in range(NSB):
        drain(s)
        if s + 1 < NSB:
            fire(s + 1)
        ub, ib = ubufs[s % 2], ibufs[s % 2]

        def group_body(g, carry, ub=ub, ib=ib, s=s):
            acc = jnp.zeros((L,), jnp.float32)
            for k in range(L):
                b = g * L + k
                p = jnp.zeros((L,), jnp.float32)
                for f in range(0, D, L):
                    u = ub[b // 8, b % 8, pl.ds(f, L)]
                    v = ib[b // 8, b % 8, pl.ds(f, L)]
                    p = p + u * v
                acc = jnp.where(lane == k, jnp.sum(p), acc)
            oacc[pl.ds(s * CH + g * L, L)] = acc
            return carry

        lax.fori_loop(0, CH // L, group_body, 0)

    pltpu.sync_copy(oacc, out_hbm.at[pl.ds(base, BPW)])


def kernel(user, item, user_factors, item_factors):
    utab = user_factors.reshape(12500, 8, D)
    itab = item_factors.reshape(12500, 8, D)
    return _mf_sc(user.astype(jnp.int32), item.astype(jnp.int32), utab, itab)


# speedup vs baseline: 1.0948x; 1.0107x over previous
import functools

import jax
import jax.numpy as jnp
from jax import lax
from jax.experimental import pallas as pl
from jax.experimental.pallas import tpu as pltpu
from jax.experimental.pallas import tpu_sc as plsc

B = 16384
D = 64
L = 16
NC = 2
NS = 16
NW = NC * NS
BPW = B // NW
CH = 128
NSB = BPW // CH

_mesh = plsc.VectorSubcoreMesh(core_axis_name="c", subcore_axis_name="s")


@functools.partial(
    pl.kernel,
    mesh=_mesh,
    compiler_params=pltpu.CompilerParams(needs_layout_passes=False),
    out_type=jax.ShapeDtypeStruct((B,), jnp.float32),
    scratch_types=[
        pltpu.VMEM((NSB, CH), jnp.int32),
        pltpu.VMEM((NSB, CH), jnp.int32),
        pltpu.VMEM((CH // 8, 8, D), jnp.float32),
        pltpu.VMEM((CH // 8, 8, D), jnp.float32),
        pltpu.VMEM((CH // 8, 8, D), jnp.float32),
        pltpu.VMEM((CH // 8, 8, D), jnp.float32),
        pltpu.VMEM((BPW,), jnp.float32),
        pltpu.SemaphoreType.DMA,
        pltpu.SemaphoreType.DMA,
        pltpu.SemaphoreType.DMA,
        pltpu.SemaphoreType.DMA,
        pltpu.SemaphoreType.DMA,
    ],
)
def _mf_sc(user_hbm, item_hbm, utab_hbm, itab_hbm, out_hbm,
           uidx, iidx, ub0, ub1, ib0, ib1, oacc, us0, us1, is0, is1, isem):
    wid = lax.axis_index("s") * NC + lax.axis_index("c")
    base = wid * BPW
    ubufs, ibufs = (ub0, ub1), (ib0, ib1)
    usems, isems = (us0, us1), (is0, is1)

    idx_copies = []
    for j in range(NSB):
        idx_copies.append(pltpu.async_copy(
            user_hbm.at[pl.ds(base + j * CH, CH)], uidx.at[j], isem))
        idx_copies.append(pltpu.async_copy(
            item_hbm.at[pl.ds(base + j * CH, CH)], iidx.at[j], isem))
    for c in idx_copies:
        c.wait()

    def fire(s):
        p = s % 2

        def g_body(g, carry):
            uvec = uidx[s, pl.ds(g * L, L)]
            ivec = iidx[s, pl.ds(g * L, L)]
            for k in range(L):
                m = g * L + k
                ur, ir = uvec[k], ivec[k]
                pltpu.async_copy(
                    utab_hbm.at[pl.ds(ur >> 3, 1), ur & 7, :],
                    ubufs[p].at[pl.ds(m // 8, 1), m % 8, :], usems[p])
                pltpu.async_copy(
                    itab_hbm.at[pl.ds(ir >> 3, 1), ir & 7, :],
                    ibufs[p].at[pl.ds(m // 8, 1), m % 8, :], isems[p])
            return carry

        lax.fori_loop(0, CH // L, g_body, 0)

    def drain(s):
        p = s % 2
        pltpu.make_async_copy(utab_hbm.at[pl.ds(0, CH // 8), :, :], ubufs[p],
                              usems[p]).wait()
        pltpu.make_async_copy(itab_hbm.at[pl.ds(0, CH // 8), :, :], ibufs[p],
                              isems[p]).wait()

    lane = lax.broadcasted_iota(jnp.int32, (L,), 0)

    fire(0)
    for s in range(NSB):
        if s + 1 < NSB:
            fire(s + 1)
        drain(s)
        ub, ib = ubufs[s % 2], ibufs[s % 2]

        def group_body(g, carry, ub=ub, ib=ib, s=s):
            acc = jnp.zeros((L,), jnp.float32)
            for k in range(L):
                b = g * L + k
                p = jnp.zeros((L,), jnp.float32)
                for f in range(0, D, L):
                    u = ub[b // 8, b % 8, pl.ds(f, L)]
                    v = ib[b // 8, b % 8, pl.ds(f, L)]
                    p = p + u * v
                acc = jnp.where(lane == k, jnp.sum(p), acc)
            oacc[pl.ds(s * CH + g * L, L)] = acc
            return carry

        lax.fori_loop(0, CH // L, group_body, 0)

    pltpu.sync_copy(oacc, out_hbm.at[pl.ds(base, BPW)])


def kernel(user, item, user_factors, item_factors):
    utab = user_factors.reshape(12500, 8, D)
    itab = item_factors.reshape(12500, 8, D)
    return _mf_sc(user.astype(jnp.int32), item.astype(jnp.int32), utab, itab)


# speedup vs baseline: 1.1313x; 1.0333x over previous
import functools

import jax
import jax.numpy as jnp
from jax import lax
from jax.experimental import pallas as pl
from jax.experimental.pallas import tpu as pltpu
from jax.experimental.pallas import tpu_sc as plsc

B = 16384
D = 64
L = 16
NC = 2
NS = 16
NW = NC * NS
BPW = B // NW
CH = 128
NSB = BPW // CH

_mesh = plsc.VectorSubcoreMesh(core_axis_name="c", subcore_axis_name="s")


@functools.partial(
    pl.kernel,
    mesh=_mesh,
    compiler_params=pltpu.CompilerParams(needs_layout_passes=False),
    out_type=jax.ShapeDtypeStruct((B,), jnp.float32),
    scratch_types=[
        pltpu.VMEM((NSB, CH), jnp.int32),
        pltpu.VMEM((NSB, CH), jnp.int32),
        pltpu.VMEM((CH // 8, 8, D), jnp.float32),
        pltpu.VMEM((CH // 8, 8, D), jnp.float32),
        pltpu.VMEM((CH, D), jnp.float32),
        pltpu.VMEM((CH, D), jnp.float32),
        pltpu.VMEM((BPW,), jnp.float32),
        pltpu.SemaphoreType.DMA,
        pltpu.SemaphoreType.DMA,
        pltpu.SemaphoreType.DMA,
        pltpu.SemaphoreType.DMA,
        pltpu.SemaphoreType.DMA,
    ],
)
def _mf_sc(user_hbm, item_hbm, utab_hbm, itab_hbm, out_hbm,
           uidx, iidx, ub0, ub1, ib0, ib1, oacc, us0, us1, is0, is1, isem):
    wid = lax.axis_index("s") * NC + lax.axis_index("c")
    base = wid * BPW
    ubufs, ibufs = (ub0, ub1), (ib0, ib1)
    usems, isems = (us0, us1), (is0, is1)

    idx_copies = []
    for j in range(NSB):
        idx_copies.append(pltpu.async_copy(
            user_hbm.at[pl.ds(base + j * CH, CH)], uidx.at[j], isem))
        idx_copies.append(pltpu.async_copy(
            item_hbm.at[pl.ds(base + j * CH, CH)], iidx.at[j], isem))
    for c in idx_copies:
        c.wait()

    def fire(s):
        p = s % 2

        def g_body(g, carry):
            uvec = uidx[s, pl.ds(g * L, L)]
            ivec = iidx[s, pl.ds(g * L, L)]
            for k in range(L):
                m = g * L + k
                ur, ir = uvec[k], ivec[k]
                pltpu.async_copy(
                    utab_hbm.at[pl.ds(ur >> 3, 1), ur & 7, :],
                    ubufs[p].at[pl.ds(m // 8, 1), m % 8, :], usems[p])
                pltpu.async_copy(
                    itab_hbm.at[pl.ds(ir, 1), :],
                    ibufs[p].at[pl.ds(m, 1), :], isems[p])
            return carry

        lax.fori_loop(0, CH // L, g_body, 0)

    def drain(s):
        p = s % 2
        pltpu.make_async_copy(utab_hbm.at[pl.ds(0, CH // 8), :, :], ubufs[p],
                              usems[p]).wait()
        pltpu.make_async_copy(itab_hbm.at[pl.ds(0, CH), :], ibufs[p],
                              isems[p]).wait()

    lane = lax.broadcasted_iota(jnp.int32, (L,), 0)

    fire(0)
    for s in range(NSB):
        if s + 1 < NSB:
            fire(s + 1)
        drain(s)
        ub, ib = ubufs[s % 2], ibufs[s % 2]

        def group_body(g, carry, ub=ub, ib=ib, s=s):
            acc = jnp.zeros((L,), jnp.float32)
            for k in range(L):
                b = g * L + k
                p = jnp.zeros((L,), jnp.float32)
                for f in range(0, D, L):
                    u = ub[b // 8, b % 8, pl.ds(f, L)]
                    v = ib[b, pl.ds(f, L)]
                    p = p + u * v
                acc = jnp.where(lane == k, jnp.sum(p), acc)
            oacc[pl.ds(s * CH + g * L, L)] = acc
            return carry

        lax.fori_loop(0, CH // L, group_body, 0)

    pltpu.sync_copy(oacc, out_hbm.at[pl.ds(base, BPW)])


def kernel(user, item, user_factors, item_factors):
    utab = user_factors.reshape(12500, 8, D)
    return _mf_sc(user.astype(jnp.int32), item.astype(jnp.int32), utab,
                  item_factors)
